# Initial kernel scaffold; baseline (speedup 1.0000x reference)
import jax
import jax.numpy as jnp
from jax.experimental import pallas as pl


def kernel(h, x, edges, edge_attr, params):
    raise NotImplementedError("write your pallas kernel here")


# speedup vs baseline: 1.0070x; 1.0070x over previous
import functools
import jax
import jax.numpy as jnp
from jax.experimental import pallas as pl
from jax.experimental.pallas import tpu as pltpu

H = 128
XW = 16

_INTERPRET = False


def _silu(v):
    return v * (1.0 / (1.0 + jnp.exp(-v)))


def _prologue_body(h0_ref, wemb_ref, bemb_ref, w1a_ref, w1b_ref, b1_ref,
                   h_ref, a_ref, b_ref):
    h = jnp.dot(h0_ref[...], wemb_ref[...],
                preferred_element_type=jnp.float32) + bemb_ref[...]
    h_ref[...] = h
    a_ref[...] = jnp.dot(h, w1a_ref[...],
                         preferred_element_type=jnp.float32) + b1_ref[...]
    b_ref[...] = jnp.dot(h, w1b_ref[...], preferred_element_type=jnp.float32)


def _edge_body(g_ref, cd_ref, ea_ref, wr_ref, wea_ref, w2_ref, b2_ref, ef_ref):
    cd = cd_ref[...]
    radial = jnp.sum(cd[:, :3] * cd[:, :3], axis=1, keepdims=True)
    mpre = g_ref[...] + radial * wr_ref[...]
    mpre = mpre + jnp.dot(ea_ref[...], wea_ref[...],
                          preferred_element_type=jnp.float32)
    m = _silu(mpre)
    ef_ref[...] = _silu(jnp.dot(m, w2_ref[...],
                                preferred_element_type=jnp.float32) + b2_ref[...])


def _node_body(h_ref, ph_ref, pc_ref, x_ref, w3a_ref, w3b_ref, b3_ref,
               w4_ref, b4_ref, wna_ref, wnb_ref, bn_ref,
               h_ref_out, x_ref_out, a_ref_out, b_ref_out):
    h = h_ref[...]
    aggh = jnp.sum(ph_ref[...], axis=0)
    aggc = jnp.sum(pc_ref[...], axis=0)
    cnt = jnp.clip(aggc[:, 3:4], 1.0, None)
    lane = jax.lax.broadcasted_iota(jnp.int32, aggc.shape, 1)
    aggc = jnp.where(lane < 3, aggc, 0.0)
    x_ref_out[...] = x_ref[...] + aggc / cnt
    t = _silu(jnp.dot(h, w3a_ref[...], preferred_element_type=jnp.float32)
              + jnp.dot(aggh, w3b_ref[...], preferred_element_type=jnp.float32)
              + b3_ref[...])
    hn = h + jnp.dot(t, w4_ref[...],
                     preferred_element_type=jnp.float32) + b4_ref[...]
    h_ref_out[...] = hn
    a_ref_out[...] = jnp.dot(hn, wna_ref[...],
                             preferred_element_type=jnp.float32) + bn_ref[...]
    b_ref_out[...] = jnp.dot(hn, wnb_ref[...], preferred_element_type=jnp.float32)


def _node_final_body(h_ref, ph_ref, pc_ref, x_ref, w3a_ref, w3b_ref, b3_ref,
                     w4_ref, b4_ref, wout_ref, bout_ref,
                     h_ref_out, x_ref_out):
    h = h_ref[...]
    aggh = jnp.sum(ph_ref[...], axis=0)
    aggc = jnp.sum(pc_ref[...], axis=0)
    cnt = jnp.clip(aggc[:, 3:4], 1.0, None)
    lane = jax.lax.broadcasted_iota(jnp.int32, aggc.shape, 1)
    aggc = jnp.where(lane < 3, aggc, 0.0)
    x_ref_out[...] = x_ref[...] + aggc / cnt
    t = _silu(jnp.dot(h, w3a_ref[...], preferred_element_type=jnp.float32)
              + jnp.dot(aggh, w3b_ref[...], preferred_element_type=jnp.float32)
              + b3_ref[...])
    hn = h + jnp.dot(t, w4_ref[...],
                     preferred_element_type=jnp.float32) + b4_ref[...]
    h_ref_out[...] = jnp.dot(hn, wout_ref[...],
                             preferred_element_type=jnp.float32) + bout_ref[...]


def _full(shape):
    return pl.BlockSpec(shape, lambda i: tuple(0 for _ in shape))


def _run_prologue(h0, wemb, bemb, w1a, w1b, b1):
    n = h0.shape[0]
    bn = 2000
    grid = (n // bn,)
    row = pl.BlockSpec((bn, H), lambda i: (i, 0))
    out_sd = jax.ShapeDtypeStruct((n, H), jnp.float32)
    return pl.pallas_call(
        _prologue_body,
        grid=grid,
        in_specs=[row, _full((H, H)), _full((1, H)), _full((H, H)),
                  _full((H, H)), _full((1, H))],
        out_specs=[row, row, row],
        out_shape=[out_sd, out_sd, out_sd],
        interpret=_INTERPRET,
    )(h0, wemb, bemb, w1a, w1b, b1)


def _run_edge(g, cd16, ea8, wr, wea, w2, b2):
    e = g.shape[0]
    be = 5000
    grid = (e // be,)
    return pl.pallas_call(
        _edge_body,
        grid=grid,
        in_specs=[pl.BlockSpec((be, H), lambda i: (i, 0)),
                  pl.BlockSpec((be, XW), lambda i: (i, 0)),
                  pl.BlockSpec((be, 8), lambda i: (i, 0)),
                  _full((1, H)), _full((8, H)), _full((H, H)), _full((1, H))],
        out_specs=pl.BlockSpec((be, H), lambda i: (i, 0)),
        out_shape=jax.ShapeDtypeStruct((e, H), jnp.float32),
        interpret=_INTERPRET,
    )(g, cd16, ea8, wr, wea, w2, b2)


def _run_node(h, ph, pc, x16, w3a, w3b, b3, w4, b4, wna, wnb, bn_):
    n = h.shape[0]
    p = ph.shape[0]
    bn = 2000
    grid = (n // bn,)
    row = pl.BlockSpec((bn, H), lambda i: (i, 0))
    rowx = pl.BlockSpec((bn, XW), lambda i: (i, 0))
    out_sd = jax.ShapeDtypeStruct((n, H), jnp.float32)
    return pl.pallas_call(
        _node_body,
        grid=grid,
        in_specs=[row,
                  pl.BlockSpec((p, bn, H), lambda i: (0, i, 0)),
                  pl.BlockSpec((p, bn, XW), lambda i: (0, i, 0)),
                  rowx,
                  _full((H, H)), _full((H, H)), _full((1, H)),
                  _full((H, H)), _full((1, H)),
                  _full((H, H)), _full((H, H)), _full((1, H))],
        out_specs=[row, rowx, row, row],
        out_shape=[out_sd, jax.ShapeDtypeStruct((n, XW), jnp.float32),
                   out_sd, out_sd],
        interpret=_INTERPRET,
    )(h, ph, pc, x16, w3a, w3b, b3, w4, b4, wna, wnb, bn_)


def _run_node_final(h, ph, pc, x16, w3a, w3b, b3, w4, b4, wout, bout):
    n = h.shape[0]
    p = ph.shape[0]
    bn = 2000
    grid = (n // bn,)
    row = pl.BlockSpec((bn, H), lambda i: (i, 0))
    rowx = pl.BlockSpec((bn, XW), lambda i: (i, 0))
    return pl.pallas_call(
        _node_final_body,
        grid=grid,
        in_specs=[row,
                  pl.BlockSpec((p, bn, H), lambda i: (0, i, 0)),
                  pl.BlockSpec((p, bn, XW), lambda i: (0, i, 0)),
                  rowx,
                  _full((H, H)), _full((H, H)), _full((1, H)),
                  _full((H, H)), _full((1, H)),
                  _full((H, H)), _full((1, H))],
        out_specs=[row, rowx],
        out_shape=[jax.ShapeDtypeStruct((n, H), jnp.float32),
                   jax.ShapeDtypeStruct((n, XW), jnp.float32)],
        interpret=_INTERPRET,
    )(h, ph, pc, x16, w3a, w3b, b3, w4, b4, wout, bout)


def _gather_stage(a, b, x16, row, col):
    g = jnp.take(a, row, axis=0) + jnp.take(b, col, axis=0)
    cd16 = jnp.take(x16, row, axis=0) - jnp.take(x16, col, axis=0)
    lane = jax.lax.broadcasted_iota(jnp.int32, cd16.shape, 1)
    cd16 = jnp.where(lane == 3, 1.0, cd16)
    return g, cd16


def _scatter_stage(ef, cd16, row, n):
    ph = jax.ops.segment_sum(ef, row, num_segments=n)[None]
    pc = jax.ops.segment_sum(cd16, row, num_segments=n)[None]
    return ph, pc


def _prep(params):
    out = {}
    out["wemb"] = params["emb_in"]["w"]
    out["bemb"] = params["emb_in"]["b"][None, :]
    out["wout"] = params["emb_out"]["w"]
    out["bout"] = params["emb_out"]["b"][None, :]
    ls = []
    for p in params["layers"]:
        w1 = p["edge1"]["w"]
        ls.append({
            "w1a": w1[:H], "w1b": w1[H:2 * H], "wr": w1[2 * H:2 * H + 1],
            "wea": jnp.pad(w1[2 * H + 1:], ((0, 4), (0, 0))),
            "b1": p["edge1"]["b"][None, :],
            "w2": p["edge2"]["w"], "b2": p["edge2"]["b"][None, :],
            "w3a": p["node1"]["w"][:H], "w3b": p["node1"]["w"][H:],
            "b3": p["node1"]["b"][None, :],
            "w4": p["node2"]["w"], "b4": p["node2"]["b"][None, :],
        })
    out["layers"] = ls
    return out


@jax.jit
def kernel(h, x, edges, edge_attr, params):
    n = h.shape[0]
    e = edges.shape[1]
    pp = _prep(params)
    ls = pp["layers"]
    row, col = edges[0], edges[1]
    x16 = jnp.pad(x, ((0, 0), (0, XW - 3)))
    ea8 = jnp.pad(edge_attr, ((0, 0), (0, 8 - edge_attr.shape[1])))

    hcur, a, b = _run_prologue(h, pp["wemb"], pp["bemb"],
                               ls[0]["w1a"], ls[0]["w1b"], ls[0]["b1"])
    nl = len(ls)
    for i, lp in enumerate(ls):
        g, cd16 = _gather_stage(a, b, x16, row, col)
        ef = _run_edge(g, cd16, ea8, lp["wr"], lp["wea"], lp["w2"], lp["b2"])
        ph, pc = _scatter_stage(ef, cd16, row, n)
        if i + 1 < nl:
            nxt = ls[i + 1]
            hcur, x16, a, b = _run_node(
                hcur, ph, pc, x16, lp["w3a"], lp["w3b"], lp["b3"],
                lp["w4"], lp["b4"], nxt["w1a"], nxt["w1b"], nxt["b1"])
        else:
            hcur, x16 = _run_node_final(
                hcur, ph, pc, x16, lp["w3a"], lp["w3b"], lp["b3"],
                lp["w4"], lp["b4"], pp["wout"], pp["bout"])
    return hcur, x16[:, :3]


# speedup vs baseline: 2.8067x; 2.7874x over previous
import functools
import jax
import jax.numpy as jnp
from jax import lax
from jax.experimental import pallas as pl
from jax.experimental.pallas import tpu as pltpu
from jax.experimental.pallas import tpu_sc as plsc

H = 128
XW = 16

SC_NC = 2
SC_NS = 16
SC_NW = SC_NC * SC_NS
SC_W = 80

_INTERPRET = False


def _silu(v):
    return v * (1.0 / (1.0 + jnp.exp(-v)))


def _prologue_body(h0_ref, wemb_ref, bemb_ref, w1a_ref, w1b_ref, b1_ref,
                   h_ref, a_ref, b_ref):
    h = jnp.dot(h0_ref[...], wemb_ref[...],
                preferred_element_type=jnp.float32) + bemb_ref[...]
    h_ref[...] = h
    a_ref[...] = jnp.dot(h, w1a_ref[...],
                         preferred_element_type=jnp.float32) + b1_ref[...]
    b_ref[...] = jnp.dot(h, w1b_ref[...], preferred_element_type=jnp.float32)


def _edge_body(g_ref, cd_ref, ea_ref, wr_ref, wea_ref, w2_ref, b2_ref, ef_ref):
    cd = cd_ref[...]
    radial = jnp.sum(cd[:, :3] * cd[:, :3], axis=1, keepdims=True)
    mpre = g_ref[...] + radial * wr_ref[...]
    mpre = mpre + jnp.dot(ea_ref[...], wea_ref[...],
                          preferred_element_type=jnp.float32)
    m = _silu(mpre)
    ef_ref[...] = _silu(jnp.dot(m, w2_ref[...],
                                preferred_element_type=jnp.float32) + b2_ref[...])


def _node_body(h_ref, ph_ref, pc_ref, x_ref, w3a_ref, w3b_ref, b3_ref,
               w4_ref, b4_ref, wna_ref, wnb_ref, bn_ref,
               h_ref_out, x_ref_out, a_ref_out, b_ref_out):
    h = h_ref[...]
    aggh = jnp.sum(ph_ref[...], axis=0)
    aggc = jnp.sum(pc_ref[...], axis=0)
    cnt = jnp.clip(aggc[:, 3:4], 1.0, None)
    lane = jax.lax.broadcasted_iota(jnp.int32, aggc.shape, 1)
    aggc = jnp.where(lane < 3, aggc, 0.0)
    x_ref_out[...] = x_ref[...] + aggc / cnt
    t = _silu(jnp.dot(h, w3a_ref[...], preferred_element_type=jnp.float32)
              + jnp.dot(aggh, w3b_ref[...], preferred_element_type=jnp.float32)
              + b3_ref[...])
    hn = h + jnp.dot(t, w4_ref[...],
                     preferred_element_type=jnp.float32) + b4_ref[...]
    h_ref_out[...] = hn
    a_ref_out[...] = jnp.dot(hn, wna_ref[...],
                             preferred_element_type=jnp.float32) + bn_ref[...]
    b_ref_out[...] = jnp.dot(hn, wnb_ref[...], preferred_element_type=jnp.float32)


def _node_final_body(h_ref, ph_ref, pc_ref, x_ref, w3a_ref, w3b_ref, b3_ref,
                     w4_ref, b4_ref, wout_ref, bout_ref,
                     h_ref_out, x_ref_out):
    h = h_ref[...]
    aggh = jnp.sum(ph_ref[...], axis=0)
    aggc = jnp.sum(pc_ref[...], axis=0)
    cnt = jnp.clip(aggc[:, 3:4], 1.0, None)
    lane = jax.lax.broadcasted_iota(jnp.int32, aggc.shape, 1)
    aggc = jnp.where(lane < 3, aggc, 0.0)
    x_ref_out[...] = x_ref[...] + aggc / cnt
    t = _silu(jnp.dot(h, w3a_ref[...], preferred_element_type=jnp.float32)
              + jnp.dot(aggh, w3b_ref[...], preferred_element_type=jnp.float32)
              + b3_ref[...])
    hn = h + jnp.dot(t, w4_ref[...],
                     preferred_element_type=jnp.float32) + b4_ref[...]
    h_ref_out[...] = jnp.dot(hn, wout_ref[...],
                             preferred_element_type=jnp.float32) + bout_ref[...]


def _full(shape):
    return pl.BlockSpec(shape, lambda i: tuple(0 for _ in shape))


def _run_prologue(h0, wemb, bemb, w1a, w1b, b1):
    n = h0.shape[0]
    bn = 2000
    grid = (n // bn,)
    row = pl.BlockSpec((bn, H), lambda i: (i, 0))
    out_sd = jax.ShapeDtypeStruct((n, H), jnp.float32)
    return pl.pallas_call(
        _prologue_body,
        grid=grid,
        in_specs=[row, _full((H, H)), _full((1, H)), _full((H, H)),
                  _full((H, H)), _full((1, H))],
        out_specs=[row, row, row],
        out_shape=[out_sd, out_sd, out_sd],
        interpret=_INTERPRET,
    )(h0, wemb, bemb, w1a, w1b, b1)


def _run_edge(g, cd16, ea8, wr, wea, w2, b2):
    e = g.shape[0]
    be = 5000
    grid = (e // be,)
    return pl.pallas_call(
        _edge_body,
        grid=grid,
        in_specs=[pl.BlockSpec((be, H), lambda i: (i, 0)),
                  pl.BlockSpec((be, XW), lambda i: (i, 0)),
                  pl.BlockSpec((be, 8), lambda i: (i, 0)),
                  _full((1, H)), _full((8, H)), _full((H, H)), _full((1, H))],
        out_specs=pl.BlockSpec((be, H), lambda i: (i, 0)),
        out_shape=jax.ShapeDtypeStruct((e, H), jnp.float32),
        interpret=_INTERPRET,
    )(g, cd16, ea8, wr, wea, w2, b2)


def _run_node(h, ph, pc, x16, w3a, w3b, b3, w4, b4, wna, wnb, bn_):
    n = h.shape[0]
    p = ph.shape[0]
    bn = 2000
    grid = (n // bn,)
    row = pl.BlockSpec((bn, H), lambda i: (i, 0))
    rowx = pl.BlockSpec((bn, XW), lambda i: (i, 0))
    out_sd = jax.ShapeDtypeStruct((n, H), jnp.float32)
    return pl.pallas_call(
        _node_body,
        grid=grid,
        in_specs=[row,
                  pl.BlockSpec((p, bn, H), lambda i: (0, i, 0)),
                  pl.BlockSpec((p, bn, XW), lambda i: (0, i, 0)),
                  rowx,
                  _full((H, H)), _full((H, H)), _full((1, H)),
                  _full((H, H)), _full((1, H)),
                  _full((H, H)), _full((H, H)), _full((1, H))],
        out_specs=[row, rowx, row, row],
        out_shape=[out_sd, jax.ShapeDtypeStruct((n, XW), jnp.float32),
                   out_sd, out_sd],
        interpret=_INTERPRET,
    )(h, ph, pc, x16, w3a, w3b, b3, w4, b4, wna, wnb, bn_)


def _run_node_final(h, ph, pc, x16, w3a, w3b, b3, w4, b4, wout, bout):
    n = h.shape[0]
    p = ph.shape[0]
    bn = 2000
    grid = (n // bn,)
    row = pl.BlockSpec((bn, H), lambda i: (i, 0))
    rowx = pl.BlockSpec((bn, XW), lambda i: (i, 0))
    return pl.pallas_call(
        _node_final_body,
        grid=grid,
        in_specs=[row,
                  pl.BlockSpec((p, bn, H), lambda i: (0, i, 0)),
                  pl.BlockSpec((p, bn, XW), lambda i: (0, i, 0)),
                  rowx,
                  _full((H, H)), _full((H, H)), _full((1, H)),
                  _full((H, H)), _full((1, H)),
                  _full((H, H)), _full((1, H))],
        out_specs=[row, rowx],
        out_shape=[jax.ShapeDtypeStruct((n, H), jnp.float32),
                   jax.ShapeDtypeStruct((n, XW), jnp.float32)],
        interpret=_INTERPRET,
    )(h, ph, pc, x16, w3a, w3b, b3, w4, b4, wout, bout)


def _sc_mesh():
    return plsc.VectorSubcoreMesh(core_axis_name="c", subcore_axis_name="s")


def _gather_stage(a, b, x16, row3, col3):
    e = row3.shape[0] * row3.shape[1] * row3.shape[2]
    r_per_w = row3.shape[1]
    w = row3.shape[2]
    e_per_w = r_per_w * w

    @functools.partial(
        pl.kernel, mesh=_sc_mesh(),
        compiler_params=pltpu.CompilerParams(use_tc_tiling_on_sc=False),
        out_type=[jax.ShapeDtypeStruct((e, H), jnp.float32),
                  jax.ShapeDtypeStruct((e, XW), jnp.float32)],
        scratch_types=[
            pltpu.VMEM((r_per_w, w), jnp.int32),
            pltpu.VMEM((r_per_w, w), jnp.int32),
            pltpu.VMEM((w, H), jnp.float32),
            pltpu.VMEM((w, H), jnp.float32),
            pltpu.VMEM((w, XW), jnp.float32),
            pltpu.VMEM((w, XW), jnp.float32),
            pltpu.SemaphoreType.DMA,
        ],
    )
    def k(a_hbm, b_hbm, x_hbm, row_hbm, col_hbm, g_hbm, cd_hbm,
          idxr, idxc, bufa, bufb, bufxr, bufxc, sem):
        wid = lax.axis_index("s") * SC_NC + lax.axis_index("c")
        pltpu.sync_copy(row_hbm.at[wid], idxr)
        pltpu.sync_copy(col_hbm.at[wid], idxc)
        lanes = lax.iota(jnp.int32, XW)
        e3 = jnp.where(lanes == 3, 1.0, 0.0).astype(jnp.float32)

        @pl.loop(0, r_per_w)
        def _round(r):
            base = wid * e_per_w + r * w
            cpa = pltpu.async_copy(a_hbm.at[idxr.at[r]], bufa, sem)
            cpb = pltpu.async_copy(b_hbm.at[idxc.at[r]], bufb, sem)
            cpxr = pltpu.async_copy(x_hbm.at[idxr.at[r]], bufxr, sem)
            cpxc = pltpu.async_copy(x_hbm.at[idxc.at[r]], bufxc, sem)
            cpa.wait()
            cpb.wait()
            cpxr.wait()
            cpxc.wait()

            @pl.loop(0, w)
            def _rowi(i):
                for c in range(H // 16):
                    sl = (pl.ds(i, 1), pl.ds(c * 16, 16))
                    bufa.at[*sl][...] = bufa.at[*sl][...] + bufb.at[*sl][...]
                bufxr.at[i][...] = bufxr.at[i][...] - bufxc.at[i][...] + e3

            pltpu.sync_copy(bufa, g_hbm.at[pl.ds(base, w)])
            pltpu.sync_copy(bufxr, cd_hbm.at[pl.ds(base, w)])

    return k(a, b, x16, row3, col3)


def _scatter_stage(ef, cd16, row3, zh, zc, n):
    r_per_w = row3.shape[1]
    w = row3.shape[2]
    e_per_w = r_per_w * w
    n_per_s = n // SC_NS

    @functools.partial(
        pl.kernel, mesh=_sc_mesh(),
        compiler_params=pltpu.CompilerParams(use_tc_tiling_on_sc=False),
        out_type=[jax.ShapeDtypeStruct((SC_NC, n, H), jnp.float32),
                  jax.ShapeDtypeStruct((SC_NC, n, XW), jnp.float32)],
        scratch_types=[
            pltpu.VMEM((r_per_w, w), jnp.int32),
            pltpu.VMEM((w, H), jnp.float32),
            pltpu.VMEM((w, XW), jnp.float32),
            pltpu.VMEM_SHARED((n, H), jnp.float32),
            pltpu.VMEM_SHARED((n, XW), jnp.float32),
            pltpu.SemaphoreType.DMA,
        ],
    )
    def k(ef_hbm, cd_hbm, row_hbm, zh_hbm, zc_hbm, ph_hbm, pc_hbm,
          idx, buf, bufc, acc_h, acc_c, sem):
        cid = lax.axis_index("c")
        sid = lax.axis_index("s")
        wid = sid * SC_NC + cid
        nslc = pl.ds(sid * n_per_s, n_per_s)
        pltpu.sync_copy(zh_hbm.at[nslc], acc_h.at[nslc])
        pltpu.sync_copy(zc_hbm.at[nslc], acc_c.at[nslc])
        pltpu.sync_copy(row_hbm.at[wid], idx)
        plsc.subcore_barrier()

        @pl.loop(0, r_per_w)
        def _round(r):
            base = wid * e_per_w + r * w
            pltpu.sync_copy(ef_hbm.at[pl.ds(base, w)], buf)
            pltpu.sync_copy(cd_hbm.at[pl.ds(base, w)], bufc)
            pltpu.sync_copy(buf, acc_h.at[idx.at[r]], add=True)
            pltpu.sync_copy(bufc, acc_c.at[idx.at[r]], add=True)

        plsc.subcore_barrier()
        pltpu.sync_copy(acc_h.at[nslc], ph_hbm.at[cid, nslc])
        pltpu.sync_copy(acc_c.at[nslc], pc_hbm.at[cid, nslc])

    return k(ef, cd16, row3, zh, zc)


def _prep(params):
    out = {}
    out["wemb"] = params["emb_in"]["w"]
    out["bemb"] = params["emb_in"]["b"][None, :]
    out["wout"] = params["emb_out"]["w"]
    out["bout"] = params["emb_out"]["b"][None, :]
    ls = []
    for p in params["layers"]:
        w1 = p["edge1"]["w"]
        ls.append({
            "w1a": w1[:H], "w1b": w1[H:2 * H], "wr": w1[2 * H:2 * H + 1],
            "wea": jnp.pad(w1[2 * H + 1:], ((0, 4), (0, 0))),
            "b1": p["edge1"]["b"][None, :],
            "w2": p["edge2"]["w"], "b2": p["edge2"]["b"][None, :],
            "w3a": p["node1"]["w"][:H], "w3b": p["node1"]["w"][H:],
            "b3": p["node1"]["b"][None, :],
            "w4": p["node2"]["w"], "b4": p["node2"]["b"][None, :],
        })
    out["layers"] = ls
    return out


@jax.jit
def kernel(h, x, edges, edge_attr, params):
    n = h.shape[0]
    e = edges.shape[1]
    pp = _prep(params)
    ls = pp["layers"]
    row, col = edges[0], edges[1]
    e_per_w = e // SC_NW
    r_per_w = e_per_w // SC_W
    row3 = row.reshape(SC_NW, r_per_w, SC_W)
    col3 = col.reshape(SC_NW, r_per_w, SC_W)
    zh = jnp.zeros((n, H), jnp.float32)
    zc = jnp.zeros((n, XW), jnp.float32)
    x16 = jnp.pad(x, ((0, 0), (0, XW - 3)))
    ea8 = jnp.pad(edge_attr, ((0, 0), (0, 8 - edge_attr.shape[1])))

    hcur, a, b = _run_prologue(h, pp["wemb"], pp["bemb"],
                               ls[0]["w1a"], ls[0]["w1b"], ls[0]["b1"])
    nl = len(ls)
    for i, lp in enumerate(ls):
        g, cd16 = _gather_stage(a, b, x16, row3, col3)
        ef = _run_edge(g, cd16, ea8, lp["wr"], lp["wea"], lp["w2"], lp["b2"])
        ph, pc = _scatter_stage(ef, cd16, row3, zh, zc, n)
        if i + 1 < nl:
            nxt = ls[i + 1]
            hcur, x16, a, b = _run_node(
                hcur, ph, pc, x16, lp["w3a"], lp["w3b"], lp["b3"],
                lp["w4"], lp["b4"], nxt["w1a"], nxt["w1b"], nxt["b1"])
        else:
            hcur, x16 = _run_node_final(
                hcur, ph, pc, x16, lp["w3a"], lp["w3b"], lp["b3"],
                lp["w4"], lp["b4"], pp["wout"], pp["bout"])
    return hcur, x16[:, :3]


# speedup vs baseline: 3.1876x; 1.1357x over previous
import functools
import jax
import jax.numpy as jnp
from jax import lax
from jax.experimental import pallas as pl
from jax.experimental.pallas import tpu as pltpu
from jax.experimental.pallas import tpu_sc as plsc

H = 128
XW = 16

SC_NC = 2
SC_NS = 16
SC_NW = SC_NC * SC_NS
SC_W = 80

_INTERPRET = False


def _silu(v):
    return v * (1.0 / (1.0 + jnp.exp(-v)))


def _prologue_body(h0_ref, wemb_ref, bemb_ref, w1a_ref, w1b_ref, b1_ref,
                   h_ref, a_ref, b_ref):
    h = jnp.dot(h0_ref[...], wemb_ref[...],
                preferred_element_type=jnp.float32) + bemb_ref[...]
    h_ref[...] = h
    a_ref[...] = jnp.dot(h, w1a_ref[...],
                         preferred_element_type=jnp.float32) + b1_ref[...]
    b_ref[...] = jnp.dot(h, w1b_ref[...], preferred_element_type=jnp.float32)


def _edge_body(g_ref, cd_ref, ea_ref, wr_ref, wea_ref, w2_ref, b2_ref, ef_ref):
    cd = cd_ref[...]
    radial = jnp.sum(cd[:, :3] * cd[:, :3], axis=1, keepdims=True)
    mpre = g_ref[...] + radial * wr_ref[...]
    mpre = mpre + jnp.dot(ea_ref[...], wea_ref[...],
                          preferred_element_type=jnp.float32)
    m = _silu(mpre)
    ef_ref[...] = _silu(jnp.dot(m, w2_ref[...],
                                preferred_element_type=jnp.float32) + b2_ref[...])


def _node_body(h_ref, ph_ref, pc_ref, x_ref, w3a_ref, w3b_ref, b3_ref,
               w4_ref, b4_ref, wna_ref, wnb_ref, bn_ref,
               h_ref_out, x_ref_out, a_ref_out, b_ref_out):
    h = h_ref[...]
    aggh = jnp.sum(ph_ref[...], axis=0)
    aggc = jnp.sum(pc_ref[...], axis=0)
    cnt = jnp.clip(aggc[:, 3:4], 1.0, None)
    lane = jax.lax.broadcasted_iota(jnp.int32, aggc.shape, 1)
    aggc = jnp.where(lane < 3, aggc, 0.0)
    x_ref_out[...] = x_ref[...] + aggc / cnt
    t = _silu(jnp.dot(h, w3a_ref[...], preferred_element_type=jnp.float32)
              + jnp.dot(aggh, w3b_ref[...], preferred_element_type=jnp.float32)
              + b3_ref[...])
    hn = h + jnp.dot(t, w4_ref[...],
                     preferred_element_type=jnp.float32) + b4_ref[...]
    h_ref_out[...] = hn
    a_ref_out[...] = jnp.dot(hn, wna_ref[...],
                             preferred_element_type=jnp.float32) + bn_ref[...]
    b_ref_out[...] = jnp.dot(hn, wnb_ref[...], preferred_element_type=jnp.float32)


def _node_final_body(h_ref, ph_ref, pc_ref, x_ref, w3a_ref, w3b_ref, b3_ref,
                     w4_ref, b4_ref, wout_ref, bout_ref,
                     h_ref_out, x_ref_out):
    h = h_ref[...]
    aggh = jnp.sum(ph_ref[...], axis=0)
    aggc = jnp.sum(pc_ref[...], axis=0)
    cnt = jnp.clip(aggc[:, 3:4], 1.0, None)
    lane = jax.lax.broadcasted_iota(jnp.int32, aggc.shape, 1)
    aggc = jnp.where(lane < 3, aggc, 0.0)
    x_ref_out[...] = x_ref[...] + aggc / cnt
    t = _silu(jnp.dot(h, w3a_ref[...], preferred_element_type=jnp.float32)
              + jnp.dot(aggh, w3b_ref[...], preferred_element_type=jnp.float32)
              + b3_ref[...])
    hn = h + jnp.dot(t, w4_ref[...],
                     preferred_element_type=jnp.float32) + b4_ref[...]
    h_ref_out[...] = jnp.dot(hn, wout_ref[...],
                             preferred_element_type=jnp.float32) + bout_ref[...]


def _full(shape):
    return pl.BlockSpec(shape, lambda i: tuple(0 for _ in shape))


def _run_prologue(h0, wemb, bemb, w1a, w1b, b1):
    n = h0.shape[0]
    bn = 2000
    grid = (n // bn,)
    row = pl.BlockSpec((bn, H), lambda i: (i, 0))
    out_sd = jax.ShapeDtypeStruct((n, H), jnp.float32)
    return pl.pallas_call(
        _prologue_body,
        grid=grid,
        in_specs=[row, _full((H, H)), _full((1, H)), _full((H, H)),
                  _full((H, H)), _full((1, H))],
        out_specs=[row, row, row],
        out_shape=[out_sd, out_sd, out_sd],
        interpret=_INTERPRET,
    )(h0, wemb, bemb, w1a, w1b, b1)


def _run_edge(g, cd16, ea8, wr, wea, w2, b2):
    e = g.shape[0]
    be = 5000
    grid = (e // be,)
    return pl.pallas_call(
        _edge_body,
        grid=grid,
        in_specs=[pl.BlockSpec((be, H), lambda i: (i, 0)),
                  pl.BlockSpec((be, XW), lambda i: (i, 0)),
                  pl.BlockSpec((be, 8), lambda i: (i, 0)),
                  _full((1, H)), _full((8, H)), _full((H, H)), _full((1, H))],
        out_specs=pl.BlockSpec((be, H), lambda i: (i, 0)),
        out_shape=jax.ShapeDtypeStruct((e, H), jnp.float32),
        interpret=_INTERPRET,
    )(g, cd16, ea8, wr, wea, w2, b2)


def _run_node(h, ph, pc, x16, w3a, w3b, b3, w4, b4, wna, wnb, bn_):
    n = h.shape[0]
    p = ph.shape[0]
    bn = 2000
    grid = (n // bn,)
    row = pl.BlockSpec((bn, H), lambda i: (i, 0))
    rowx = pl.BlockSpec((bn, XW), lambda i: (i, 0))
    out_sd = jax.ShapeDtypeStruct((n, H), jnp.float32)
    return pl.pallas_call(
        _node_body,
        grid=grid,
        in_specs=[row,
                  pl.BlockSpec((p, bn, H), lambda i: (0, i, 0)),
                  pl.BlockSpec((p, bn, XW), lambda i: (0, i, 0)),
                  rowx,
                  _full((H, H)), _full((H, H)), _full((1, H)),
                  _full((H, H)), _full((1, H)),
                  _full((H, H)), _full((H, H)), _full((1, H))],
        out_specs=[row, rowx, row, row],
        out_shape=[out_sd, jax.ShapeDtypeStruct((n, XW), jnp.float32),
                   out_sd, out_sd],
        interpret=_INTERPRET,
    )(h, ph, pc, x16, w3a, w3b, b3, w4, b4, wna, wnb, bn_)


def _run_node_final(h, ph, pc, x16, w3a, w3b, b3, w4, b4, wout, bout):
    n = h.shape[0]
    p = ph.shape[0]
    bn = 2000
    grid = (n // bn,)
    row = pl.BlockSpec((bn, H), lambda i: (i, 0))
    rowx = pl.BlockSpec((bn, XW), lambda i: (i, 0))
    return pl.pallas_call(
        _node_final_body,
        grid=grid,
        in_specs=[row,
                  pl.BlockSpec((p, bn, H), lambda i: (0, i, 0)),
                  pl.BlockSpec((p, bn, XW), lambda i: (0, i, 0)),
                  rowx,
                  _full((H, H)), _full((H, H)), _full((1, H)),
                  _full((H, H)), _full((1, H)),
                  _full((H, H)), _full((1, H))],
        out_specs=[row, rowx],
        out_shape=[jax.ShapeDtypeStruct((n, H), jnp.float32),
                   jax.ShapeDtypeStruct((n, XW), jnp.float32)],
        interpret=_INTERPRET,
    )(h, ph, pc, x16, w3a, w3b, b3, w4, b4, wout, bout)


def _sc_mesh():
    return plsc.VectorSubcoreMesh(core_axis_name="c", subcore_axis_name="s")


def _gather_stage(a, b, x16, row3, col3):
    e = row3.shape[0] * row3.shape[1] * row3.shape[2]
    r_per_w = row3.shape[1]
    w = row3.shape[2]
    e_per_w = r_per_w * w

    @functools.partial(
        pl.kernel, mesh=_sc_mesh(),
        compiler_params=pltpu.CompilerParams(use_tc_tiling_on_sc=False),
        out_type=[jax.ShapeDtypeStruct((e, H), jnp.float32),
                  jax.ShapeDtypeStruct((e, XW), jnp.float32)],
        scratch_types=[
            pltpu.VMEM((r_per_w, w), jnp.int32),
            pltpu.VMEM((r_per_w, w), jnp.int32),
            pltpu.VMEM((w, H), jnp.float32),
            pltpu.VMEM((w, H), jnp.float32),
            pltpu.VMEM((w, XW), jnp.float32),
            pltpu.VMEM((w, XW), jnp.float32),
            pltpu.SemaphoreType.DMA,
        ],
    )
    def k(a_hbm, b_hbm, x_hbm, row_hbm, col_hbm, g_hbm, cd_hbm,
          idxr, idxc, bufa, bufb, bufxr, bufxc, sem):
        wid = lax.axis_index("s") * SC_NC + lax.axis_index("c")
        pltpu.sync_copy(row_hbm.at[wid], idxr)
        pltpu.sync_copy(col_hbm.at[wid], idxc)
        lanes = lax.iota(jnp.int32, XW)
        e3 = jnp.where(lanes == 3, 1.0, 0.0).astype(jnp.float32)

        @pl.loop(0, r_per_w)
        def _round(r):
            base = wid * e_per_w + r * w
            cpa = pltpu.async_copy(a_hbm.at[idxr.at[r]], bufa, sem)
            cpb = pltpu.async_copy(b_hbm.at[idxc.at[r]], bufb, sem)
            cpxr = pltpu.async_copy(x_hbm.at[idxr.at[r]], bufxr, sem)
            cpxc = pltpu.async_copy(x_hbm.at[idxc.at[r]], bufxc, sem)
            cpa.wait()
            cpb.wait()
            cpxr.wait()
            cpxc.wait()

            @pl.loop(0, w)
            def _rowi(i):
                for c in range(H // 16):
                    sl = (pl.ds(i, 1), pl.ds(c * 16, 16))
                    bufa.at[*sl][...] = bufa.at[*sl][...] + bufb.at[*sl][...]
                bufxr.at[i][...] = bufxr.at[i][...] - bufxc.at[i][...] + e3

            pltpu.sync_copy(bufa, g_hbm.at[pl.ds(base, w)])
            pltpu.sync_copy(bufxr, cd_hbm.at[pl.ds(base, w)])

    return k(a, b, x16, row3, col3)


def _scatter_stage(ef, cd16, row3, zh, zc, n):
    r_per_w = row3.shape[1]
    w = row3.shape[2]
    e_per_w = r_per_w * w
    n_per_s = n // SC_NS

    @functools.partial(
        pl.kernel, mesh=_sc_mesh(),
        compiler_params=pltpu.CompilerParams(use_tc_tiling_on_sc=False),
        out_type=[jax.ShapeDtypeStruct((SC_NC, n, H), jnp.float32),
                  jax.ShapeDtypeStruct((SC_NC, n, XW), jnp.float32)],
        scratch_types=[
            pltpu.VMEM((r_per_w, w), jnp.int32),
            pltpu.VMEM((2, w, H), jnp.float32),
            pltpu.VMEM((2, w, XW), jnp.float32),
            pltpu.VMEM_SHARED((n, H), jnp.float32),
            pltpu.VMEM_SHARED((n, XW), jnp.float32),
            pltpu.SemaphoreType.DMA,
            pltpu.SemaphoreType.DMA,
            pltpu.SemaphoreType.DMA,
        ],
    )
    def k(ef_hbm, cd_hbm, row_hbm, zh_hbm, zc_hbm, ph_hbm, pc_hbm,
          idx, buf, bufc, acc_h, acc_c, semf0, semf1, sems):
        cid = lax.axis_index("c")
        sid = lax.axis_index("s")
        wid = sid * SC_NC + cid
        nslc = pl.ds(sid * n_per_s, n_per_s)
        pltpu.sync_copy(zh_hbm.at[nslc], acc_h.at[nslc])
        pltpu.sync_copy(zc_hbm.at[nslc], acc_c.at[nslc])
        pltpu.sync_copy(row_hbm.at[wid], idx)
        plsc.subcore_barrier()

        semf = (semf0, semf1)
        base0 = wid * e_per_w
        last = r_per_w - 1

        def start_fetch(r, b):
            base = base0 + r * w
            pltpu.async_copy(ef_hbm.at[pl.ds(base, w)], buf.at[b], semf[b])
            pltpu.async_copy(cd_hbm.at[pl.ds(base, w)], bufc.at[b], semf[b])

        def wait_fetch(b):
            pltpu.make_async_copy(ef_hbm.at[pl.ds(base0, w)], buf.at[b],
                                  semf[b]).wait()
            pltpu.make_async_copy(cd_hbm.at[pl.ds(base0, w)], bufc.at[b],
                                  semf[b]).wait()

        def do_scatter(r, b):
            s1 = pltpu.async_copy(buf.at[b], acc_h.at[idx.at[r]], sems,
                                  add=True)
            s2 = pltpu.async_copy(bufc.at[b], acc_c.at[idx.at[r]], sems,
                                  add=True)
            s1.wait()
            s2.wait()

        start_fetch(0, 0)
        start_fetch(1, 1)

        @pl.loop(0, last, step=2)
        def _round(g):
            for b in range(2):
                r = g + b
                wait_fetch(b)
                do_scatter(r, b)
                start_fetch(jnp.minimum(r + 2, last), b)

        wait_fetch(0)
        do_scatter(last, 0)
        wait_fetch(1)

        plsc.subcore_barrier()
        pltpu.sync_copy(acc_h.at[nslc], ph_hbm.at[cid, nslc])
        pltpu.sync_copy(acc_c.at[nslc], pc_hbm.at[cid, nslc])

    return k(ef, cd16, row3, zh, zc)


def _prep(params):
    out = {}
    out["wemb"] = params["emb_in"]["w"]
    out["bemb"] = params["emb_in"]["b"][None, :]
    out["wout"] = params["emb_out"]["w"]
    out["bout"] = params["emb_out"]["b"][None, :]
    ls = []
    for p in params["layers"]:
        w1 = p["edge1"]["w"]
        ls.append({
            "w1a": w1[:H], "w1b": w1[H:2 * H], "wr": w1[2 * H:2 * H + 1],
            "wea": jnp.pad(w1[2 * H + 1:], ((0, 4), (0, 0))),
            "b1": p["edge1"]["b"][None, :],
            "w2": p["edge2"]["w"], "b2": p["edge2"]["b"][None, :],
            "w3a": p["node1"]["w"][:H], "w3b": p["node1"]["w"][H:],
            "b3": p["node1"]["b"][None, :],
            "w4": p["node2"]["w"], "b4": p["node2"]["b"][None, :],
        })
    out["layers"] = ls
    return out


@jax.jit
def kernel(h, x, edges, edge_attr, params):
    n = h.shape[0]
    e = edges.shape[1]
    pp = _prep(params)
    ls = pp["layers"]
    row, col = edges[0], edges[1]
    e_per_w = e // SC_NW
    r_per_w = e_per_w // SC_W
    row3 = row.reshape(SC_NW, r_per_w, SC_W)
    col3 = col.reshape(SC_NW, r_per_w, SC_W)
    zh = jnp.zeros((n, H), jnp.float32)
    zc = jnp.zeros((n, XW), jnp.float32)
    x16 = jnp.pad(x, ((0, 0), (0, XW - 3)))
    ea8 = jnp.pad(edge_attr, ((0, 0), (0, 8 - edge_attr.shape[1])))

    hcur, a, b = _run_prologue(h, pp["wemb"], pp["bemb"],
                               ls[0]["w1a"], ls[0]["w1b"], ls[0]["b1"])
    nl = len(ls)
    for i, lp in enumerate(ls):
        g, cd16 = _gather_stage(a, b, x16, row3, col3)
        ef = _run_edge(g, cd16, ea8, lp["wr"], lp["wea"], lp["w2"], lp["b2"])
        ph, pc = _scatter_stage(ef, cd16, row3, zh, zc, n)
        if i + 1 < nl:
            nxt = ls[i + 1]
            hcur, x16, a, b = _run_node(
                hcur, ph, pc, x16, lp["w3a"], lp["w3b"], lp["b3"],
                lp["w4"], lp["b4"], nxt["w1a"], nxt["w1b"], nxt["b1"])
        else:
            hcur, x16 = _run_node_final(
                hcur, ph, pc, x16, lp["w3a"], lp["w3b"], lp["b3"],
                lp["w4"], lp["b4"], pp["wout"], pp["bout"])
    return hcur, x16[:, :3]


# speedup vs baseline: 4.7254x; 1.4824x over previous
import functools
import jax
import jax.numpy as jnp
from jax import lax
from jax.experimental import pallas as pl
from jax.experimental.pallas import tpu as pltpu
from jax.experimental.pallas import tpu_sc as plsc

H = 128
XW = 16

SC_NC = 2
SC_NS = 16
SC_NW = SC_NC * SC_NS
SC_W = 80

_INTERPRET = False


def _silu(v):
    return v * (1.0 / (1.0 + jnp.exp(-v)))


def _prologue_body(h0_ref, wemb_ref, bemb_ref, w1a_ref, w1b_ref, b1_ref,
                   h_ref, a_ref, b_ref):
    h = jnp.dot(h0_ref[...], wemb_ref[...],
                preferred_element_type=jnp.float32) + bemb_ref[...]
    h_ref[...] = h
    a_ref[...] = jnp.dot(h, w1a_ref[...],
                         preferred_element_type=jnp.float32) + b1_ref[...]
    b_ref[...] = jnp.dot(h, w1b_ref[...], preferred_element_type=jnp.float32)


def _edge_body(ga_ref, gb_ref, cd_ref, ea_ref, wr_ref, wea_ref, w2_ref, b2_ref,
               ef_ref):
    cd = cd_ref[...]
    radial = jnp.sum(cd[:, :3] * cd[:, :3], axis=1, keepdims=True)
    mpre = ga_ref[...] + gb_ref[...] + radial * wr_ref[...]
    mpre = mpre + jnp.dot(ea_ref[...], wea_ref[...],
                          preferred_element_type=jnp.float32)
    m = _silu(mpre)
    ef_ref[...] = _silu(jnp.dot(m, w2_ref[...],
                                preferred_element_type=jnp.float32) + b2_ref[...])


def _node_body(h_ref, ph_ref, pc_ref, x_ref, w3a_ref, w3b_ref, b3_ref,
               w4_ref, b4_ref, wna_ref, wnb_ref, bn_ref,
               h_ref_out, x_ref_out, a_ref_out, b_ref_out):
    h = h_ref[...]
    aggh = jnp.sum(ph_ref[...], axis=0)
    aggc = jnp.sum(pc_ref[...], axis=0)
    cnt = jnp.clip(aggc[:, 3:4], 1.0, None)
    lane = jax.lax.broadcasted_iota(jnp.int32, aggc.shape, 1)
    aggc = jnp.where(lane < 3, aggc, 0.0)
    x_ref_out[...] = x_ref[...] + aggc / cnt
    t = _silu(jnp.dot(h, w3a_ref[...], preferred_element_type=jnp.float32)
              + jnp.dot(aggh, w3b_ref[...], preferred_element_type=jnp.float32)
              + b3_ref[...])
    hn = h + jnp.dot(t, w4_ref[...],
                     preferred_element_type=jnp.float32) + b4_ref[...]
    h_ref_out[...] = hn
    a_ref_out[...] = jnp.dot(hn, wna_ref[...],
                             preferred_element_type=jnp.float32) + bn_ref[...]
    b_ref_out[...] = jnp.dot(hn, wnb_ref[...], preferred_element_type=jnp.float32)


def _node_final_body(h_ref, ph_ref, pc_ref, x_ref, w3a_ref, w3b_ref, b3_ref,
                     w4_ref, b4_ref, wout_ref, bout_ref,
                     h_ref_out, x_ref_out):
    h = h_ref[...]
    aggh = jnp.sum(ph_ref[...], axis=0)
    aggc = jnp.sum(pc_ref[...], axis=0)
    cnt = jnp.clip(aggc[:, 3:4], 1.0, None)
    lane = jax.lax.broadcasted_iota(jnp.int32, aggc.shape, 1)
    aggc = jnp.where(lane < 3, aggc, 0.0)
    x_ref_out[...] = x_ref[...] + aggc / cnt
    t = _silu(jnp.dot(h, w3a_ref[...], preferred_element_type=jnp.float32)
              + jnp.dot(aggh, w3b_ref[...], preferred_element_type=jnp.float32)
              + b3_ref[...])
    hn = h + jnp.dot(t, w4_ref[...],
                     preferred_element_type=jnp.float32) + b4_ref[...]
    h_ref_out[...] = jnp.dot(hn, wout_ref[...],
                             preferred_element_type=jnp.float32) + bout_ref[...]


def _full(shape):
    return pl.BlockSpec(shape, lambda i: tuple(0 for _ in shape))


def _run_prologue(h0, wemb, bemb, w1a, w1b, b1):
    n = h0.shape[0]
    bn = 2000
    grid = (n // bn,)
    row = pl.BlockSpec((bn, H), lambda i: (i, 0))
    out_sd = jax.ShapeDtypeStruct((n, H), jnp.float32)
    return pl.pallas_call(
        _prologue_body,
        grid=grid,
        in_specs=[row, _full((H, H)), _full((1, H)), _full((H, H)),
                  _full((H, H)), _full((1, H))],
        out_specs=[row, row, row],
        out_shape=[out_sd, out_sd, out_sd],
        interpret=_INTERPRET,
    )(h0, wemb, bemb, w1a, w1b, b1)


def _run_edge(ga, gb, cd16, ea8, wr, wea, w2, b2):
    e = ga.shape[0]
    be = 5000
    grid = (e // be,)
    return pl.pallas_call(
        _edge_body,
        grid=grid,
        in_specs=[pl.BlockSpec((be, H), lambda i: (i, 0)),
                  pl.BlockSpec((be, H), lambda i: (i, 0)),
                  pl.BlockSpec((be, XW), lambda i: (i, 0)),
                  pl.BlockSpec((be, 8), lambda i: (i, 0)),
                  _full((1, H)), _full((8, H)), _full((H, H)), _full((1, H))],
        out_specs=pl.BlockSpec((be, H), lambda i: (i, 0)),
        out_shape=jax.ShapeDtypeStruct((e, H), jnp.float32),
        interpret=_INTERPRET,
    )(ga, gb, cd16, ea8, wr, wea, w2, b2)


def _run_node(h, ph, pc, x16, w3a, w3b, b3, w4, b4, wna, wnb, bn_):
    n = h.shape[0]
    p = ph.shape[0]
    bn = 2000
    grid = (n // bn,)
    row = pl.BlockSpec((bn, H), lambda i: (i, 0))
    rowx = pl.BlockSpec((bn, XW), lambda i: (i, 0))
    out_sd = jax.ShapeDtypeStruct((n, H), jnp.float32)
    return pl.pallas_call(
        _node_body,
        grid=grid,
        in_specs=[row,
                  pl.BlockSpec((p, bn, H), lambda i: (0, i, 0)),
                  pl.BlockSpec((p, bn, XW), lambda i: (0, i, 0)),
                  rowx,
                  _full((H, H)), _full((H, H)), _full((1, H)),
                  _full((H, H)), _full((1, H)),
                  _full((H, H)), _full((H, H)), _full((1, H))],
        out_specs=[row, rowx, row, row],
        out_shape=[out_sd, jax.ShapeDtypeStruct((n, XW), jnp.float32),
                   out_sd, out_sd],
        interpret=_INTERPRET,
    )(h, ph, pc, x16, w3a, w3b, b3, w4, b4, wna, wnb, bn_)


def _run_node_final(h, ph, pc, x16, w3a, w3b, b3, w4, b4, wout, bout):
    n = h.shape[0]
    p = ph.shape[0]
    bn = 2000
    grid = (n // bn,)
    row = pl.BlockSpec((bn, H), lambda i: (i, 0))
    rowx = pl.BlockSpec((bn, XW), lambda i: (i, 0))
    return pl.pallas_call(
        _node_final_body,
        grid=grid,
        in_specs=[row,
                  pl.BlockSpec((p, bn, H), lambda i: (0, i, 0)),
                  pl.BlockSpec((p, bn, XW), lambda i: (0, i, 0)),
                  rowx,
                  _full((H, H)), _full((H, H)), _full((1, H)),
                  _full((H, H)), _full((1, H)),
                  _full((H, H)), _full((1, H))],
        out_specs=[row, rowx],
        out_shape=[jax.ShapeDtypeStruct((n, H), jnp.float32),
                   jax.ShapeDtypeStruct((n, XW), jnp.float32)],
        interpret=_INTERPRET,
    )(h, ph, pc, x16, w3a, w3b, b3, w4, b4, wout, bout)


def _sc_mesh():
    return plsc.VectorSubcoreMesh(core_axis_name="c", subcore_axis_name="s")


def _gather_stage(a, b, x16, row3, col3):
    e = row3.shape[0] * row3.shape[1] * row3.shape[2]
    r_per_w = row3.shape[1]
    w = row3.shape[2]
    e_per_w = r_per_w * w
    nb = 4
    assert r_per_w % nb == 1 and r_per_w > nb

    @functools.partial(
        pl.kernel, mesh=_sc_mesh(),
        compiler_params=pltpu.CompilerParams(use_tc_tiling_on_sc=False),
        out_type=[jax.ShapeDtypeStruct((e, H), jnp.float32),
                  jax.ShapeDtypeStruct((e, H), jnp.float32),
                  jax.ShapeDtypeStruct((e, XW), jnp.float32)],
        scratch_types=[
            pltpu.VMEM((r_per_w, w), jnp.int32),
            pltpu.VMEM((r_per_w, w), jnp.int32),
            pltpu.VMEM((nb, w, H), jnp.float32),
            pltpu.VMEM((nb, w, H), jnp.float32),
            pltpu.VMEM((nb, w, XW), jnp.float32),
            pltpu.VMEM((nb, w, XW), jnp.float32),
        ] + [pltpu.SemaphoreType.DMA] * (2 * nb),
    )
    def k(a_hbm, b_hbm, x_hbm, row_hbm, col_hbm, ga_hbm, gb_hbm, cd_hbm,
          idxr, idxc, bufa, bufb, bufxr, bufxc, *sems):
        semg = sems[:nb]
        semw = sems[nb:]
        wid = lax.axis_index("s") * SC_NC + lax.axis_index("c")
        pltpu.sync_copy(row_hbm.at[wid], idxr)
        pltpu.sync_copy(col_hbm.at[wid], idxc)
        lanes = lax.iota(jnp.int32, XW)
        e3 = jnp.where(lanes == 3, 1.0, 0.0).astype(jnp.float32)
        base0 = wid * e_per_w
        last = r_per_w - 1

        def fire_gathers(r, s):
            pltpu.async_copy(a_hbm.at[idxr.at[r]], bufa.at[s], semg[s])
            pltpu.async_copy(b_hbm.at[idxc.at[r]], bufb.at[s], semg[s])
            pltpu.async_copy(x_hbm.at[idxr.at[r]], bufxr.at[s], semg[s])
            pltpu.async_copy(x_hbm.at[idxc.at[r]], bufxc.at[s], semg[s])

        def wait_gathers(s):
            pltpu.make_async_copy(a_hbm.at[pl.ds(0, w)], bufa.at[s],
                                  semg[s]).wait()
            pltpu.make_async_copy(b_hbm.at[pl.ds(0, w)], bufb.at[s],
                                  semg[s]).wait()
            pltpu.make_async_copy(x_hbm.at[pl.ds(0, w)], bufxr.at[s],
                                  semg[s]).wait()
            pltpu.make_async_copy(x_hbm.at[pl.ds(0, w)], bufxc.at[s],
                                  semg[s]).wait()

        def fire_writes(r, s):
            base = base0 + r * w
            pltpu.async_copy(bufa.at[s], ga_hbm.at[pl.ds(base, w)], semw[s])
            pltpu.async_copy(bufb.at[s], gb_hbm.at[pl.ds(base, w)], semw[s])
            pltpu.async_copy(bufxr.at[s], cd_hbm.at[pl.ds(base, w)], semw[s])

        def wait_writes(s):
            pltpu.make_async_copy(bufa.at[s], ga_hbm.at[pl.ds(0, w)],
                                  semw[s]).wait()
            pltpu.make_async_copy(bufb.at[s], gb_hbm.at[pl.ds(0, w)],
                                  semw[s]).wait()
            pltpu.make_async_copy(bufxr.at[s], cd_hbm.at[pl.ds(0, w)],
                                  semw[s]).wait()

        def body(r, s):
            prev = (s - 1) % nb
            wait_writes(prev)
            fire_gathers(jnp.minimum(r + nb - 1, last), prev)
            wait_gathers(s)

            @pl.loop(0, w)
            def _rowi(i):
                bufxr.at[s, i][...] = (bufxr.at[s, i][...]
                                       - bufxc.at[s, i][...] + e3)

            fire_writes(r, s)

        for s in range(nb - 1):
            fire_gathers(s, s)
        fire_writes(0, nb - 1)

        @pl.loop(0, last, step=nb)
        def _round(g):
            for s in range(nb):
                body(g + s, s)

        body(last, 0)
        wait_writes(0)
        for s in range(1, nb):
            wait_gathers(s)

    return k(a, b, x16, row3, col3)


def _scatter_stage(ef, cd16, row3, zh, zc, n):
    r_per_w = row3.shape[1]
    w = row3.shape[2]
    e_per_w = r_per_w * w
    n_per_s = n // SC_NS

    @functools.partial(
        pl.kernel, mesh=_sc_mesh(),
        compiler_params=pltpu.CompilerParams(use_tc_tiling_on_sc=False),
        out_type=[jax.ShapeDtypeStruct((SC_NC, n, H), jnp.float32),
                  jax.ShapeDtypeStruct((SC_NC, n, XW), jnp.float32)],
        scratch_types=[
            pltpu.VMEM((r_per_w, w), jnp.int32),
            pltpu.VMEM((2, w, H), jnp.float32),
            pltpu.VMEM((2, w, XW), jnp.float32),
            pltpu.VMEM_SHARED((n, H), jnp.float32),
            pltpu.VMEM_SHARED((n, XW), jnp.float32),
            pltpu.SemaphoreType.DMA,
            pltpu.SemaphoreType.DMA,
            pltpu.SemaphoreType.DMA,
        ],
    )
    def k(ef_hbm, cd_hbm, row_hbm, zh_hbm, zc_hbm, ph_hbm, pc_hbm,
          idx, buf, bufc, acc_h, acc_c, semf0, semf1, sems):
        cid = lax.axis_index("c")
        sid = lax.axis_index("s")
        wid = sid * SC_NC + cid
        nslc = pl.ds(sid * n_per_s, n_per_s)
        pltpu.sync_copy(zh_hbm.at[nslc], acc_h.at[nslc])
        pltpu.sync_copy(zc_hbm.at[nslc], acc_c.at[nslc])
        pltpu.sync_copy(row_hbm.at[wid], idx)
        plsc.subcore_barrier()

        semf = (semf0, semf1)
        base0 = wid * e_per_w
        last = r_per_w - 1

        def start_fetch(r, b):
            base = base0 + r * w
            pltpu.async_copy(ef_hbm.at[pl.ds(base, w)], buf.at[b], semf[b])
            pltpu.async_copy(cd_hbm.at[pl.ds(base, w)], bufc.at[b], semf[b])

        def wait_fetch(b):
            pltpu.make_async_copy(ef_hbm.at[pl.ds(base0, w)], buf.at[b],
                                  semf[b]).wait()
            pltpu.make_async_copy(cd_hbm.at[pl.ds(base0, w)], bufc.at[b],
                                  semf[b]).wait()

        def do_scatter(r, b):
            s1 = pltpu.async_copy(buf.at[b], acc_h.at[idx.at[r]], sems,
                                  add=True)
            s2 = pltpu.async_copy(bufc.at[b], acc_c.at[idx.at[r]], sems,
                                  add=True)
            s1.wait()
            s2.wait()

        start_fetch(0, 0)
        start_fetch(1, 1)

        @pl.loop(0, last, step=2)
        def _round(g):
            for b in range(2):
                r = g + b
                wait_fetch(b)
                do_scatter(r, b)
                start_fetch(jnp.minimum(r + 2, last), b)

        wait_fetch(0)
        do_scatter(last, 0)
        wait_fetch(1)

        plsc.subcore_barrier()
        pltpu.sync_copy(acc_h.at[nslc], ph_hbm.at[cid, nslc])
        pltpu.sync_copy(acc_c.at[nslc], pc_hbm.at[cid, nslc])

    return k(ef, cd16, row3, zh, zc)


def _prep(params):
    out = {}
    out["wemb"] = params["emb_in"]["w"]
    out["bemb"] = params["emb_in"]["b"][None, :]
    out["wout"] = params["emb_out"]["w"]
    out["bout"] = params["emb_out"]["b"][None, :]
    ls = []
    for p in params["layers"]:
        w1 = p["edge1"]["w"]
        ls.append({
            "w1a": w1[:H], "w1b": w1[H:2 * H], "wr": w1[2 * H:2 * H + 1],
            "wea": jnp.pad(w1[2 * H + 1:], ((0, 4), (0, 0))),
            "b1": p["edge1"]["b"][None, :],
            "w2": p["edge2"]["w"], "b2": p["edge2"]["b"][None, :],
            "w3a": p["node1"]["w"][:H], "w3b": p["node1"]["w"][H:],
            "b3": p["node1"]["b"][None, :],
            "w4": p["node2"]["w"], "b4": p["node2"]["b"][None, :],
        })
    out["layers"] = ls
    return out


@jax.jit
def kernel(h, x, edges, edge_attr, params):
    n = h.shape[0]
    e = edges.shape[1]
    pp = _prep(params)
    ls = pp["layers"]
    row, col = edges[0], edges[1]
    e_per_w = e // SC_NW
    r_per_w = e_per_w // SC_W
    row3 = row.reshape(SC_NW, r_per_w, SC_W)
    col3 = col.reshape(SC_NW, r_per_w, SC_W)
    zh = jnp.zeros((n, H), jnp.float32)
    zc = jnp.zeros((n, XW), jnp.float32)
    x16 = jnp.pad(x, ((0, 0), (0, XW - 3)))
    ea8 = jnp.pad(edge_attr, ((0, 0), (0, 8 - edge_attr.shape[1])))

    hcur, a, b = _run_prologue(h, pp["wemb"], pp["bemb"],
                               ls[0]["w1a"], ls[0]["w1b"], ls[0]["b1"])
    nl = len(ls)
    for i, lp in enumerate(ls):
        ga, gb, cd16 = _gather_stage(a, b, x16, row3, col3)
        ef = _run_edge(ga, gb, cd16, ea8, lp["wr"], lp["wea"], lp["w2"],
                       lp["b2"])
        ph, pc = _scatter_stage(ef, cd16, row3, zh, zc, n)
        if i + 1 < nl:
            nxt = ls[i + 1]
            hcur, x16, a, b = _run_node(
                hcur, ph, pc, x16, lp["w3a"], lp["w3b"], lp["b3"],
                lp["w4"], lp["b4"], nxt["w1a"], nxt["w1b"], nxt["b1"])
        else:
            hcur, x16 = _run_node_final(
                hcur, ph, pc, x16, lp["w3a"], lp["w3b"], lp["b3"],
                lp["w4"], lp["b4"], pp["wout"], pp["bout"])
    return hcur, x16[:, :3]
